# baseline (speedup 1.0000x reference)
import functools

import jax
import jax.numpy as jnp
from jax import lax
from jax.experimental import pallas as pl
from jax.experimental.pallas import tpu as pltpu
from jax.experimental.pallas import tpu_sc as plsc

N_NODES = 10000
D = 128
CHUNK = 128
NC, NS = 2, 16
NW = NC * NS
ACC_ROWS = 10240
ZERO_ROWS = 64
SUB_OUT = 632
OUT_ROWS = NS * SUB_OUT
TC_BLK = 1000


def _sc_segment_sum(feature, src_idx, dst_idx, n_chunks):
    mesh = plsc.VectorSubcoreMesh(core_axis_name="c", subcore_axis_name="s")

    @functools.partial(
        pl.kernel,
        mesh=mesh,
        out_type=jax.ShapeDtypeStruct((NC, OUT_ROWS, D), jnp.float32),
        scratch_types=[
            pltpu.VMEM((n_chunks, CHUNK), jnp.int32),
            pltpu.VMEM((n_chunks, CHUNK), jnp.int32),
            pltpu.VMEM((CHUNK, D), jnp.float32),
            pltpu.VMEM((ZERO_ROWS, D), jnp.float32),
            pltpu.VMEM_SHARED((ACC_ROWS, D), jnp.float32),
            pltpu.SemaphoreType.DMA,
        ],
    )
    def k(src_hbm, dst_hbm, feat_hbm, out_hbm, src_v, dst_v, rows_a,
          zero_v, acc, semg_a):
        c = lax.axis_index("c")
        s = lax.axis_index("s")
        w = c * NS + s

        def zrow(i, _):
            for j in range(D // 16):
                zero_v[i, pl.ds(j * 16, 16)] = jnp.zeros((16,), jnp.float32)
            return 0
        lax.fori_loop(0, ZERO_ROWS, zrow, 0)

        rows_per_sub = ACC_ROWS // NS
        def zacc(i, _):
            pltpu.sync_copy(
                zero_v,
                acc.at[pl.ds(s * rows_per_sub + i * ZERO_ROWS, ZERO_ROWS)])
            return 0
        lax.fori_loop(0, rows_per_sub // ZERO_ROWS, zacc, 0)
        plsc.subcore_barrier()

        pltpu.sync_copy(src_hbm.at[w], src_v)
        pltpu.sync_copy(dst_hbm.at[w], dst_v)

        def chunk(j, _):
            pltpu.async_copy(feat_hbm.at[src_v.at[j]], rows_a, semg_a).wait()
            pltpu.sync_copy(rows_a, acc.at[dst_v.at[j]], add=True)
            return 0
        lax.fori_loop(0, n_chunks, chunk, 0)
        plsc.subcore_barrier()

        pltpu.sync_copy(acc.at[pl.ds(s * SUB_OUT, SUB_OUT)],
                        out_hbm.at[c, pl.ds(s * SUB_OUT, SUB_OUT)])

    return k(src_idx, dst_idx, feature)


def _tc_linear(partials, w_t, b2d):
    def body(p_ref, w_ref, b_ref, o_ref):
        h = p_ref[0] + p_ref[1]
        o_ref[...] = jnp.dot(h, w_ref[...],
                             preferred_element_type=jnp.float32) + b_ref[...]

    return pl.pallas_call(
        body,
        grid=(N_NODES // TC_BLK,),
        in_specs=[
            pl.BlockSpec((NC, TC_BLK, D), lambda i: (0, i, 0)),
            pl.BlockSpec((D, D), lambda i: (0, 0)),
            pl.BlockSpec((1, D), lambda i: (0, 0)),
        ],
        out_specs=pl.BlockSpec((TC_BLK, D), lambda i: (i, 0)),
        out_shape=jax.ShapeDtypeStruct((N_NODES, D), jnp.float32),
    )(partials, w_t, b2d)


def kernel(feature, edge_index, W, b):
    src = edge_index[0].astype(jnp.int32)
    dst = edge_index[1].astype(jnp.int32)
    e = src.shape[0]
    n_chunks = -(-e // (NW * CHUNK))
    per_w = e // NW
    pad_w = n_chunks * CHUNK - per_w
    n_abs = ACC_ROWS - N_NODES
    pad_dst = N_NODES + ((jnp.arange(NW, dtype=jnp.int32)[:, None] * 97
                          + jnp.arange(pad_w, dtype=jnp.int32)[None, :])
                         % n_abs)
    pad_src = ((jnp.arange(NW, dtype=jnp.int32)[:, None] * 997
                + jnp.arange(pad_w, dtype=jnp.int32)[None, :] * 31)
               % N_NODES)
    src = jnp.concatenate([src.reshape(NW, per_w), pad_src], axis=1)
    dst = jnp.concatenate(
        [dst.reshape(NW, per_w), pad_dst], axis=1)
    src = src.reshape(NW, n_chunks, CHUNK)
    dst = dst.reshape(NW, n_chunks, CHUNK)

    partials = _sc_segment_sum(feature, src, dst, n_chunks)
    return _tc_linear(partials, W.T, b.reshape(1, D))


# speedup vs baseline: 1.0127x; 1.0127x over previous
import functools

import jax
import jax.numpy as jnp
from jax import lax
from jax.experimental import pallas as pl
from jax.experimental.pallas import tpu as pltpu
from jax.experimental.pallas import tpu_sc as plsc

N_NODES = 10000
D = 128
CHUNK = 128
NC, NS = 2, 16
NW = NC * NS
ACC_ROWS = 10240
ZERO_ROWS = 64
SUB_OUT = 632
OUT_ROWS = NS * SUB_OUT
TC_BLK = 2000


def _sc_segment_sum(feature, src_idx, dst_idx, n_chunks):
    mesh = plsc.VectorSubcoreMesh(core_axis_name="c", subcore_axis_name="s")

    @functools.partial(
        pl.kernel,
        mesh=mesh,
        out_type=jax.ShapeDtypeStruct((NC, OUT_ROWS, D), jnp.float32),
        scratch_types=[
            pltpu.VMEM((n_chunks, CHUNK), jnp.int32),
            pltpu.VMEM((n_chunks, CHUNK), jnp.int32),
            pltpu.VMEM((CHUNK, D), jnp.float32),
            pltpu.VMEM((ZERO_ROWS, D), jnp.float32),
            pltpu.VMEM_SHARED((ACC_ROWS, D), jnp.float32),
            pltpu.SemaphoreType.DMA,
        ],
    )
    def k(src_hbm, dst_hbm, feat_hbm, out_hbm, src_v, dst_v, rows_a,
          zero_v, acc, semg_a):
        c = lax.axis_index("c")
        s = lax.axis_index("s")
        w = c * NS + s

        def zrow(i, _):
            for j in range(D // 16):
                zero_v[i, pl.ds(j * 16, 16)] = jnp.zeros((16,), jnp.float32)
            return 0
        lax.fori_loop(0, ZERO_ROWS, zrow, 0)

        rows_per_sub = ACC_ROWS // NS
        def zacc(i, _):
            pltpu.sync_copy(
                zero_v,
                acc.at[pl.ds(s * rows_per_sub + i * ZERO_ROWS, ZERO_ROWS)])
            return 0
        lax.fori_loop(0, rows_per_sub // ZERO_ROWS, zacc, 0)
        plsc.subcore_barrier()

        pltpu.sync_copy(src_hbm.at[w], src_v)
        pltpu.sync_copy(dst_hbm.at[w], dst_v)

        def chunk(j, _):
            pltpu.async_copy(feat_hbm.at[src_v.at[j]], rows_a, semg_a).wait()
            pltpu.sync_copy(rows_a, acc.at[dst_v.at[j]], add=True)
            return 0
        lax.fori_loop(0, n_chunks, chunk, 0)
        plsc.subcore_barrier()

        pltpu.sync_copy(acc.at[pl.ds(s * SUB_OUT, SUB_OUT)],
                        out_hbm.at[c, pl.ds(s * SUB_OUT, SUB_OUT)])

    return k(src_idx, dst_idx, feature)


def _tc_linear(partials, w_t, b2d):
    def body(p_ref, w_ref, b_ref, o_ref):
        h = p_ref[0] + p_ref[1]
        o_ref[...] = jnp.dot(h, w_ref[...],
                             preferred_element_type=jnp.float32) + b_ref[...]

    return pl.pallas_call(
        body,
        grid=(N_NODES // TC_BLK,),
        in_specs=[
            pl.BlockSpec((NC, TC_BLK, D), lambda i: (0, i, 0)),
            pl.BlockSpec((D, D), lambda i: (0, 0)),
            pl.BlockSpec((1, D), lambda i: (0, 0)),
        ],
        out_specs=pl.BlockSpec((TC_BLK, D), lambda i: (i, 0)),
        out_shape=jax.ShapeDtypeStruct((N_NODES, D), jnp.float32),
    )(partials, w_t, b2d)


def kernel(feature, edge_index, W, b):
    src = edge_index[0].astype(jnp.int32)
    dst = edge_index[1].astype(jnp.int32)
    e = src.shape[0]
    n_chunks = -(-e // (NW * CHUNK))
    per_w = e // NW
    pad_w = n_chunks * CHUNK - per_w
    n_abs = ACC_ROWS - N_NODES
    pad_dst = N_NODES + ((jnp.arange(NW, dtype=jnp.int32)[:, None] * 97
                          + jnp.arange(pad_w, dtype=jnp.int32)[None, :])
                         % n_abs)
    pad_src = ((jnp.arange(NW, dtype=jnp.int32)[:, None] * 997
                + jnp.arange(pad_w, dtype=jnp.int32)[None, :] * 31)
               % N_NODES)
    src = jnp.concatenate([src.reshape(NW, per_w), pad_src], axis=1)
    dst = jnp.concatenate(
        [dst.reshape(NW, per_w), pad_dst], axis=1)
    src = src.reshape(NW, n_chunks, CHUNK)
    dst = dst.reshape(NW, n_chunks, CHUNK)

    partials = _sc_segment_sum(feature, src, dst, n_chunks)
    return _tc_linear(partials, W.T, b.reshape(1, D))


# speedup vs baseline: 1.0219x; 1.0091x over previous
import functools

import jax
import jax.numpy as jnp
from jax import lax
from jax.experimental import pallas as pl
from jax.experimental.pallas import tpu as pltpu
from jax.experimental.pallas import tpu_sc as plsc

N_NODES = 10000
D = 128
CHUNK = 128
NC, NS = 2, 16
NW = NC * NS
ACC_ROWS = 10240
ZERO_ROWS = 64
SUB_OUT = 632
OUT_ROWS = NS * SUB_OUT
TC_BLK = 5000


def _sc_segment_sum(feature, src_idx, dst_idx, n_chunks):
    mesh = plsc.VectorSubcoreMesh(core_axis_name="c", subcore_axis_name="s")

    @functools.partial(
        pl.kernel,
        mesh=mesh,
        out_type=jax.ShapeDtypeStruct((NC, OUT_ROWS, D), jnp.float32),
        scratch_types=[
            pltpu.VMEM((n_chunks, CHUNK), jnp.int32),
            pltpu.VMEM((n_chunks, CHUNK), jnp.int32),
            pltpu.VMEM((CHUNK, D), jnp.float32),
            pltpu.VMEM((ZERO_ROWS, D), jnp.float32),
            pltpu.VMEM_SHARED((ACC_ROWS, D), jnp.float32),
            pltpu.SemaphoreType.DMA,
        ],
    )
    def k(src_hbm, dst_hbm, feat_hbm, out_hbm, src_v, dst_v, rows_a,
          zero_v, acc, semg_a):
        c = lax.axis_index("c")
        s = lax.axis_index("s")
        w = c * NS + s

        def zrow(i, _):
            for j in range(D // 16):
                zero_v[i, pl.ds(j * 16, 16)] = jnp.zeros((16,), jnp.float32)
            return 0
        lax.fori_loop(0, ZERO_ROWS, zrow, 0)

        rows_per_sub = ACC_ROWS // NS
        def zacc(i, _):
            pltpu.sync_copy(
                zero_v,
                acc.at[pl.ds(s * rows_per_sub + i * ZERO_ROWS, ZERO_ROWS)])
            return 0
        lax.fori_loop(0, rows_per_sub // ZERO_ROWS, zacc, 0)
        plsc.subcore_barrier()

        pltpu.sync_copy(src_hbm.at[w], src_v)
        pltpu.sync_copy(dst_hbm.at[w], dst_v)

        def chunk(j, _):
            pltpu.async_copy(feat_hbm.at[src_v.at[j]], rows_a, semg_a).wait()
            pltpu.sync_copy(rows_a, acc.at[dst_v.at[j]], add=True)
            return 0
        lax.fori_loop(0, n_chunks, chunk, 0)
        plsc.subcore_barrier()

        pltpu.sync_copy(acc.at[pl.ds(s * SUB_OUT, SUB_OUT)],
                        out_hbm.at[c, pl.ds(s * SUB_OUT, SUB_OUT)])

    return k(src_idx, dst_idx, feature)


def _tc_linear(partials, w_t, b2d):
    def body(p_ref, w_ref, b_ref, o_ref):
        h = p_ref[0] + p_ref[1]
        o_ref[...] = jnp.dot(h, w_ref[...],
                             preferred_element_type=jnp.float32) + b_ref[...]

    return pl.pallas_call(
        body,
        grid=(N_NODES // TC_BLK,),
        in_specs=[
            pl.BlockSpec((NC, TC_BLK, D), lambda i: (0, i, 0)),
            pl.BlockSpec((D, D), lambda i: (0, 0)),
            pl.BlockSpec((1, D), lambda i: (0, 0)),
        ],
        out_specs=pl.BlockSpec((TC_BLK, D), lambda i: (i, 0)),
        out_shape=jax.ShapeDtypeStruct((N_NODES, D), jnp.float32),
    )(partials, w_t, b2d)


def kernel(feature, edge_index, W, b):
    src = edge_index[0].astype(jnp.int32)
    dst = edge_index[1].astype(jnp.int32)
    e = src.shape[0]
    n_chunks = -(-e // (NW * CHUNK))
    per_w = e // NW
    pad_w = n_chunks * CHUNK - per_w
    n_abs = ACC_ROWS - N_NODES
    pad_dst = N_NODES + ((jnp.arange(NW, dtype=jnp.int32)[:, None] * 97
                          + jnp.arange(pad_w, dtype=jnp.int32)[None, :])
                         % n_abs)
    pad_src = ((jnp.arange(NW, dtype=jnp.int32)[:, None] * 997
                + jnp.arange(pad_w, dtype=jnp.int32)[None, :] * 31)
               % N_NODES)
    src = jnp.concatenate([src.reshape(NW, per_w), pad_src], axis=1)
    dst = jnp.concatenate(
        [dst.reshape(NW, per_w), pad_dst], axis=1)
    src = src.reshape(NW, n_chunks, CHUNK)
    dst = dst.reshape(NW, n_chunks, CHUNK)

    partials = _sc_segment_sum(feature, src, dst, n_chunks)
    return _tc_linear(partials, W.T, b.reshape(1, D))
